# baseline (speedup 1.0000x reference)
import jax
import jax.numpy as jnp
from jax.experimental import pallas as pl

K = 1024
D = 512
BN = 768


def _vq_kernel(z_ref, cbt_ref, cb_ref, esq_ref, out_ref):
    zb = z_ref[...]
    cbt = cbt_ref[...]
    dots = jax.lax.dot_general(
        zb, cbt, (((1,), (0,)), ((), ())),
        preferred_element_type=jnp.float32)
    scores = esq_ref[...] - 2.0 * dots
    idx = jnp.argmin(scores, axis=1)
    oh = (jax.lax.broadcasted_iota(jnp.int32, scores.shape, 1)
          == idx[:, None]).astype(jnp.float32)
    q = jax.lax.dot_general(
        oh, cb_ref[...], (((1,), (0,)), ((), ())),
        preferred_element_type=jnp.float32)
    out_ref[...] = q


def kernel(z, codebook):
    B, T, Dd = z.shape
    zf = z.reshape(-1, Dd)
    n = zf.shape[0]
    e_sq = jnp.sum(codebook * codebook, axis=1)[None, :]
    grid = (n // BN,)
    out = pl.pallas_call(
        _vq_kernel,
        grid=grid,
        in_specs=[
            pl.BlockSpec((BN, Dd), lambda i: (i, 0)),
            pl.BlockSpec((Dd, K), lambda i: (0, 0)),
            pl.BlockSpec((K, Dd), lambda i: (0, 0)),
            pl.BlockSpec((1, K), lambda i: (0, 0)),
        ],
        out_specs=pl.BlockSpec((BN, Dd), lambda i: (i, 0)),
        out_shape=jax.ShapeDtypeStruct((n, Dd), jnp.float32),
    )(zf, codebook.T, codebook, e_sq)
    return out.reshape(B, T, Dd)


# speedup vs baseline: 1.0396x; 1.0396x over previous
import jax
import jax.numpy as jnp
from jax.experimental import pallas as pl

K = 1024
D = 512
BN = 768


def _vq_kernel(z_ref, cbt_ref, cb_ref, out_ref):
    zb = z_ref[...]
    cbt = cbt_ref[...]
    dots = jax.lax.dot_general(
        zb, cbt, (((1,), (0,)), ((), ())),
        preferred_element_type=jnp.float32)
    e_sq = jnp.sum(cbt * cbt, axis=0, keepdims=True)
    scores = e_sq - 2.0 * dots
    idx = jnp.argmin(scores, axis=1)
    oh = (jax.lax.broadcasted_iota(jnp.int32, scores.shape, 1)
          == idx[:, None]).astype(jnp.float32)
    q = jax.lax.dot_general(
        oh, cb_ref[...], (((1,), (0,)), ((), ())),
        preferred_element_type=jnp.float32)
    out_ref[...] = q


def kernel(z, codebook):
    B, T, Dd = z.shape
    zf = z.reshape(-1, Dd)
    n = zf.shape[0]
    grid = (n // BN,)
    out = pl.pallas_call(
        _vq_kernel,
        grid=grid,
        in_specs=[
            pl.BlockSpec((BN, Dd), lambda i: (i, 0)),
            pl.BlockSpec((Dd, K), lambda i: (0, 0)),
            pl.BlockSpec((K, Dd), lambda i: (0, 0)),
        ],
        out_specs=pl.BlockSpec((BN, Dd), lambda i: (i, 0)),
        out_shape=jax.ShapeDtypeStruct((n, Dd), jnp.float32),
    )(zf, codebook.T, codebook)
    return out.reshape(B, T, Dd)


# speedup vs baseline: 1.0907x; 1.0491x over previous
import jax
import jax.numpy as jnp
from jax.experimental import pallas as pl

K = 1024
D = 512
BN = 768


def _vq_kernel(z_ref, cbt_ref, cb_ref, out_ref):
    zb = z_ref[...]
    cbt = cbt_ref[...]
    dots = jax.lax.dot_general(
        zb, cbt, (((1,), (0,)), ((), ())),
        preferred_element_type=jnp.float32)
    e_sq = jnp.sum(cbt * cbt, axis=0, keepdims=True)
    scores = e_sq - 2.0 * dots
    idx = jnp.argmin(scores, axis=1)
    oh = (jax.lax.broadcasted_iota(jnp.int32, scores.shape, 1)
          == idx[:, None]).astype(jnp.float32)
    q = jax.lax.dot_general(
        oh, cb_ref[...], (((1,), (0,)), ((), ())),
        preferred_element_type=jnp.float32)
    out_ref[...] = zb + (q - zb)


def kernel(z, codebook):
    B, T, Dd = z.shape
    zf = z.reshape(-1, Dd)
    n = zf.shape[0]
    grid = (n // BN,)
    out = pl.pallas_call(
        _vq_kernel,
        grid=grid,
        in_specs=[
            pl.BlockSpec((BN, Dd), lambda i: (i, 0)),
            pl.BlockSpec((Dd, K), lambda i: (0, 0)),
            pl.BlockSpec((K, Dd), lambda i: (0, 0)),
        ],
        out_specs=pl.BlockSpec((BN, Dd), lambda i: (i, 0)),
        out_shape=jax.ShapeDtypeStruct((n, Dd), jnp.float32),
    )(zf, codebook.T, codebook)
    return out.reshape(B, T, Dd)
